# baseline (speedup 1.0000x reference)
import jax
import jax.numpy as jnp
from jax.experimental import pallas as pl
from jax.experimental.pallas import tpu as pltpu

B, S, H, D = 8, 2048, 8, 128
S_CHUNKS = 2
CS = S // S_CHUNKS


def _kv_append_kernel(len_ref, ck, cv, kt, vt, ok, ov):
    b = pl.program_id(1)
    c = pl.program_id(0)
    ok[...] = ck[...]
    ov[...] = cv[...]
    l = len_ref[b]
    base = c * CS

    @pl.when((l >= base) & (l < base + CS))
    def _():
        r = l - base
        ok[0, pl.ds(r, 1)] = kt[pl.ds(b, 1), 0]
        ov[0, pl.ds(r, 1)] = vt[pl.ds(b, 1), 0]


def kernel(cached_key, cached_value, key_token, value_token, lengths):
    out_sds = jax.ShapeDtypeStruct((B, S, H, D), jnp.float32)
    cache_spec = pl.BlockSpec((1, CS, H, D), lambda c, b: (b, c, 0, 0))
    token_spec = pl.BlockSpec((B, 1, H, D), lambda c, b: (0, 0, 0, 0))
    new_key, new_value = pl.pallas_call(
        _kv_append_kernel,
        grid=(S_CHUNKS, B),
        in_specs=[
            pl.BlockSpec(memory_space=pltpu.SMEM),
            cache_spec,
            cache_spec,
            token_spec,
            token_spec,
        ],
        out_specs=[cache_spec, cache_spec],
        out_shape=[out_sds, out_sds],
        compiler_params=pltpu.CompilerParams(
            dimension_semantics=("parallel", "parallel"),
            vmem_limit_bytes=100 * 1024 * 1024,
        ),
    )(lengths, cached_key, cached_value, key_token, value_token)
    return (new_key, new_value)
